# baseline (speedup 1.0000x reference)
import functools

import jax
import jax.numpy as jnp
from jax import lax
from jax.experimental import pallas as pl
from jax.experimental.pallas import tpu as pltpu
from jax.experimental.pallas import tpu_sc as plsc

NC = 2
NS = 16
NW = NC * NS


def _sc_mesh():
    return plsc.VectorSubcoreMesh(
        core_axis_name="c", subcore_axis_name="s", num_cores=NC, num_subcores=NS
    )


def _make_edge_scatter(NPAD, NB, D, CH):
    NCH = NB // CH
    NU = CH // 2
    RPT = NPAD // NS

    @functools.partial(
        pl.kernel,
        out_type=jax.ShapeDtypeStruct((NC, NPAD, D), jnp.float32),
        mesh=_sc_mesh(),
        scratch_types=[
            pltpu.VMEM((2, CH, 128), jnp.int32),
            pltpu.VMEM((2, CH, 128), jnp.int32),
            pltpu.VMEM((128, D), jnp.float32),
            pltpu.VMEM((128, D), jnp.float32),
            pltpu.VMEM_SHARED((NPAD, D), jnp.float32),
            pltpu.SemaphoreType.DMA,
            pltpu.SemaphoreType.DMA,
            pltpu.SemaphoreType.DMA,
            pltpu.SemaphoreType.DMA,
            pltpu.SemaphoreType.DMA,
            pltpu.SemaphoreType.DMA,
        ],
    )
    def kern(zcat_hbm, gidx_hbm, sidx_hbm, zrow_hbm, g_out,
             gix, six, rb0, rb1, g_sh, sg0, sg1, ss0, ss1, sil0, sil1):
        c = lax.axis_index("c")
        s = lax.axis_index("s")
        wid = c * NS + s
        pltpu.sync_copy(zrow_hbm, g_sh.at[pl.ds(s * RPT, RPT)])

        def idx_load(ch, slot, sil):
            pltpu.async_copy(gidx_hbm.at[wid, pl.ds(ch * CH, CH)], gix.at[slot], sil)
            pltpu.async_copy(sidx_hbm.at[wid, pl.ds(ch * CH, CH)], six.at[slot], sil)

        def idx_wait(ch, slot, sil):
            pltpu.make_async_copy(
                gidx_hbm.at[wid, pl.ds(ch * CH, CH)], gix.at[slot], sil
            ).wait()
            pltpu.make_async_copy(
                sidx_hbm.at[wid, pl.ds(ch * CH, CH)], six.at[slot], sil
            ).wait()

        def g_start(slot, t, rb, sg):
            pltpu.async_copy(zcat_hbm.at[gix.at[slot, t]], rb, sg)

        def g_wait(slot, rb, sg):
            pltpu.make_async_copy(zcat_hbm.at[gix.at[slot, 0]], rb, sg).wait()

        def s_start(slot, t, rb, ss):
            pltpu.async_copy(rb, g_sh.at[six.at[slot, t]], ss, add=True)

        def s_wait(slot, rb, ss):
            pltpu.make_async_copy(rb, g_sh.at[six.at[slot, 0]], ss).wait()

        idx_load(0, 0, sil0)
        plsc.subcore_barrier()

        def chunk(ci, carry):
            slot = lax.rem(ci, 2)

            @pl.when(slot == 0)
            def _():
                idx_wait(ci, 0, sil0)

            @pl.when(slot == 1)
            def _():
                idx_wait(ci, 1, sil1)

            @pl.when(ci + 1 < NCH)
            def _():
                nslot = lax.rem(ci + 1, 2)

                @pl.when(nslot == 0)
                def _():
                    idx_load(ci + 1, 0, sil0)

                @pl.when(nslot == 1)
                def _():
                    idx_load(ci + 1, 1, sil1)

            g_start(slot, 0, rb0, sg0)

            def pair(u, carry2):
                t0 = 2 * u
                t1 = t0 + 1
                g_wait(slot, rb0, sg0)
                s_start(slot, t0, rb0, ss0)
                g_start(slot, t1, rb1, sg1)
                g_wait(slot, rb1, sg1)
                s_start(slot, t1, rb1, ss1)
                s_wait(slot, rb0, ss0)

                @pl.when(u < NU - 1)
                def _():
                    g_start(slot, t0 + 2, rb0, sg0)

                s_wait(slot, rb1, ss1)
                return carry2

            lax.fori_loop(0, NU, pair, 0)
            return carry

        lax.fori_loop(0, NCH, chunk, 0)
        plsc.subcore_barrier()
        pltpu.sync_copy(
            g_sh.at[pl.ds(s * RPT, RPT)], g_out.at[c, pl.ds(s * RPT, RPT)]
        )

    return kern


def _z_body(hb, pb, w1, w2, b2, out):
    out[...] = (
        jnp.dot(hb[...], w1[...], preferred_element_type=jnp.float32)
        + jnp.dot(pb[...], w2[...], preferred_element_type=jnp.float32)
        + b2[...]
    )


def _merge_body(g0, g1, nrm, wn, out):
    gg = g0[...] + g1[...]
    out[...] = jnp.dot(gg, wn[...], preferred_element_type=jnp.float32) * nrm[...]


def kernel(h, pos_enc, norm, prev_h, emb_rel, W_hp, b_hp, W_neighbor, edge_index, edge_type):
    N, D = h.shape
    P = pos_enc.shape[1]
    R = emb_rel.shape[0]
    E = edge_type.shape[0]
    B = 64
    CH = 16
    NPAD = 10240
    PP = 8
    EPW = NPAD
    EP = NW * EPW
    NB = EPW // B

    posp = jnp.concatenate([pos_enc, jnp.zeros((N, PP - P), jnp.float32)], axis=1)
    w1 = W_hp[:D]
    w2 = jnp.concatenate([W_hp[D:], jnp.zeros((PP - P, D), jnp.float32)], axis=0)
    b2 = b_hp.reshape(1, D)
    npad = jnp.zeros((EP - E,), jnp.int32)
    src3 = jnp.concatenate([edge_index[0], npad]).reshape(NW, NB, B)
    et3 = (jnp.concatenate([edge_type, npad]) + N).reshape(NW, NB, B)
    dst3 = jnp.concatenate(
        [edge_index[1], jnp.full((EP - E,), NPAD - 1, jnp.int32)]
    ).reshape(NW, NB, B)
    gidx = jnp.concatenate([src3, et3], axis=2)
    sidx = jnp.concatenate([dst3, dst3], axis=2)
    zrow = jnp.zeros((NPAD // NS, D), jnp.float32)

    BN = 1000
    z = pl.pallas_call(
        _z_body,
        grid=(N // BN,),
        in_specs=[
            pl.BlockSpec((BN, D), lambda i: (i, 0)),
            pl.BlockSpec((BN, PP), lambda i: (i, 0)),
            pl.BlockSpec((D, D), lambda i: (0, 0)),
            pl.BlockSpec((PP, D), lambda i: (0, 0)),
            pl.BlockSpec((1, D), lambda i: (0, 0)),
        ],
        out_specs=pl.BlockSpec((BN, D), lambda i: (i, 0)),
        out_shape=jax.ShapeDtypeStruct((N, D), jnp.float32),
    )(h, posp, w1, w2, b2)

    zcat = jnp.concatenate([z, emb_rel], axis=0)
    g_parts = _make_edge_scatter(NPAD, NB, D, CH)(zcat, gidx, sidx, zrow)

    node_repr = pl.pallas_call(
        _merge_body,
        grid=(N // BN,),
        in_specs=[
            pl.BlockSpec((BN, D), lambda i: (i, 0)),
            pl.BlockSpec((BN, D), lambda i: (i, 0)),
            pl.BlockSpec((BN, 1), lambda i: (i, 0)),
            pl.BlockSpec((D, D), lambda i: (0, 0)),
        ],
        out_specs=pl.BlockSpec((BN, D), lambda i: (i, 0)),
        out_shape=jax.ShapeDtypeStruct((N, D), jnp.float32),
    )(g_parts[0], g_parts[1], norm, W_neighbor)
    return node_repr, pos_enc


# speedup vs baseline: 1.0833x; 1.0833x over previous
import functools

import jax
import jax.numpy as jnp
from jax import lax
from jax.experimental import pallas as pl
from jax.experimental.pallas import tpu as pltpu
from jax.experimental.pallas import tpu_sc as plsc

NC = 2
NS = 16
NW = NC * NS


def _sc_mesh():
    return plsc.VectorSubcoreMesh(
        core_axis_name="c", subcore_axis_name="s", num_cores=NC, num_subcores=NS
    )


def _make_edge_scatter(NPAD, NB, D, CH, BR):
    NCH = NB // CH
    NJ = CH // 4
    RPT = NPAD // NS

    @functools.partial(
        pl.kernel,
        out_type=jax.ShapeDtypeStruct((NC, NPAD, D), jnp.float32),
        mesh=_sc_mesh(),
        scratch_types=[
            pltpu.VMEM((CH, BR), jnp.int32),
            pltpu.VMEM((CH, BR), jnp.int32),
            pltpu.VMEM((BR, D), jnp.float32),
            pltpu.VMEM((BR, D), jnp.float32),
            pltpu.VMEM((BR, D), jnp.float32),
            pltpu.VMEM((BR, D), jnp.float32),
            pltpu.VMEM_SHARED((NPAD, D), jnp.float32),
            pltpu.SemaphoreType.DMA,
            pltpu.SemaphoreType.DMA,
            pltpu.SemaphoreType.DMA,
            pltpu.SemaphoreType.DMA,
            pltpu.SemaphoreType.DMA,
            pltpu.SemaphoreType.DMA,
            pltpu.SemaphoreType.DMA,
            pltpu.SemaphoreType.DMA,
        ],
    )
    def kern(zcat_hbm, gidx_hbm, sidx_hbm, zrow_hbm, g_out,
             gix, six, rb0, rb1, rb2, rb3, g_sh,
             sg0, sg1, sg2, sg3, ss0, ss1, ss2, ss3):
        c = lax.axis_index("c")
        s = lax.axis_index("s")
        wid = c * NS + s
        rbs = (rb0, rb1, rb2, rb3)
        sgs = (sg0, sg1, sg2, sg3)
        sss = (ss0, ss1, ss2, ss3)
        pltpu.sync_copy(zrow_hbm, g_sh.at[pl.ds(s * RPT, RPT)])
        plsc.subcore_barrier()

        def g_start(t, k):
            pltpu.async_copy(zcat_hbm.at[gix.at[t]], rbs[k], sgs[k])

        def g_wait(k):
            pltpu.make_async_copy(zcat_hbm.at[gix.at[0]], rbs[k], sgs[k]).wait()

        def s_start(t, k):
            pltpu.async_copy(rbs[k], g_sh.at[six.at[t]], sss[k], add=True)

        def s_wait(k):
            pltpu.make_async_copy(rbs[k], g_sh.at[six.at[0]], sss[k]).wait()

        def chunk(ci, carry):
            base = ci * CH
            pltpu.sync_copy(gidx_hbm.at[wid, pl.ds(base, CH)], gix)
            pltpu.sync_copy(sidx_hbm.at[wid, pl.ds(base, CH)], six)
            g_start(0, 0)
            g_start(1, 1)

            def group(j, carry2):
                t0 = 4 * j
                for t in range(4):
                    i = t0 + t
                    g_wait(t)
                    s_start(i, t)
                    k2 = (t + 2) % 4
                    if t < 2:
                        @pl.when(j > 0)
                        def _():
                            s_wait(k2)

                        g_start(i + 2, k2)
                    else:
                        s_wait(k2)

                        @pl.when(j < NJ - 1)
                        def _():
                            g_start(i + 2, k2)
                return carry2

            lax.fori_loop(0, NJ, group, 0)
            s_wait(2)
            s_wait(3)
            return carry

        lax.fori_loop(0, NCH, chunk, 0)
        plsc.subcore_barrier()
        pltpu.sync_copy(
            g_sh.at[pl.ds(s * RPT, RPT)], g_out.at[c, pl.ds(s * RPT, RPT)]
        )

    return kern


def _z_body(hb, pb, w1, w2, b2, out):
    out[...] = (
        jnp.dot(hb[...], w1[...], preferred_element_type=jnp.float32)
        + jnp.dot(pb[...], w2[...], preferred_element_type=jnp.float32)
        + b2[...]
    )


def _merge_body(g0, g1, nrm, wn, out):
    gg = g0[...] + g1[...]
    out[...] = jnp.dot(gg, wn[...], preferred_element_type=jnp.float32) * nrm[...]


def kernel(h, pos_enc, norm, prev_h, emb_rel, W_hp, b_hp, W_neighbor, edge_index, edge_type):
    N, D = h.shape
    P = pos_enc.shape[1]
    R = emb_rel.shape[0]
    E = edge_type.shape[0]
    B = 32
    CH = 40
    NPAD = 10240
    PP = 8
    EPW = NPAD
    EP = NW * EPW
    NB = EPW // B

    posp = jnp.concatenate([pos_enc, jnp.zeros((N, PP - P), jnp.float32)], axis=1)
    w1 = W_hp[:D]
    w2 = jnp.concatenate([W_hp[D:], jnp.zeros((PP - P, D), jnp.float32)], axis=0)
    b2 = b_hp.reshape(1, D)
    npad = jnp.zeros((EP - E,), jnp.int32)
    src3 = jnp.concatenate([edge_index[0], npad]).reshape(NW, NB, B)
    et3 = (jnp.concatenate([edge_type, npad]) + N).reshape(NW, NB, B)
    dst3 = jnp.concatenate(
        [edge_index[1], jnp.full((EP - E,), NPAD - 1, jnp.int32)]
    ).reshape(NW, NB, B)
    gidx = jnp.concatenate([src3, et3], axis=2)
    sidx = jnp.concatenate([dst3, dst3], axis=2)
    zrow = jnp.zeros((NPAD // NS, D), jnp.float32)

    BN = 1000
    z = pl.pallas_call(
        _z_body,
        grid=(N // BN,),
        in_specs=[
            pl.BlockSpec((BN, D), lambda i: (i, 0)),
            pl.BlockSpec((BN, PP), lambda i: (i, 0)),
            pl.BlockSpec((D, D), lambda i: (0, 0)),
            pl.BlockSpec((PP, D), lambda i: (0, 0)),
            pl.BlockSpec((1, D), lambda i: (0, 0)),
        ],
        out_specs=pl.BlockSpec((BN, D), lambda i: (i, 0)),
        out_shape=jax.ShapeDtypeStruct((N, D), jnp.float32),
    )(h, posp, w1, w2, b2)

    zcat = jnp.concatenate([z, emb_rel], axis=0)
    g_parts = _make_edge_scatter(NPAD, NB, D, CH, 2 * B)(zcat, gidx, sidx, zrow)

    node_repr = pl.pallas_call(
        _merge_body,
        grid=(N // BN,),
        in_specs=[
            pl.BlockSpec((BN, D), lambda i: (i, 0)),
            pl.BlockSpec((BN, D), lambda i: (i, 0)),
            pl.BlockSpec((BN, 1), lambda i: (i, 0)),
            pl.BlockSpec((D, D), lambda i: (0, 0)),
        ],
        out_specs=pl.BlockSpec((BN, D), lambda i: (i, 0)),
        out_shape=jax.ShapeDtypeStruct((N, D), jnp.float32),
    )(g_parts[0], g_parts[1], norm, W_neighbor)
    return node_repr, pos_enc
